# baseline (speedup 1.0000x reference)
import jax
import jax.numpy as jnp
import numpy as np
from jax.experimental import pallas as pl

N_NODES = 10000
N_GRAPHS = 64
HIDDEN = 256
T_EMBED = 128
NUM_LAYERS = 5
MAX_K = 30
RADIUS = 1.5
NUM_BASIS = 32
EDGE_DIM = NUM_BASIS + 3


def _silu(x):
    return x * jax.nn.sigmoid(x)


def _build_radius_graph(coords, batch, r, max_k, chunk=2000):
    N = coords.shape[0]
    sq = jnp.sum(coords ** 2, axis=-1)
    src_list, mask_list = [], []
    for s in range(0, N, chunk):
        c = coords[s:s + chunk]
        b = batch[s:s + chunk]
        m = c.shape[0]
        d2 = jnp.sum(c ** 2, axis=-1)[:, None] + sq[None, :] - 2.0 * (c @ coords.T)
        same = b[:, None] == batch[None, :]
        self_m = jnp.arange(N)[None, :] == (s + jnp.arange(m))[:, None]
        valid = same & (~self_m) & (d2 <= r * r)
        d2m = jnp.where(valid, d2, jnp.inf)
        vals, idxs = jax.lax.top_k(-d2m, max_k)
        src_list.append(idxs)
        mask_list.append(vals > -jnp.inf)
    src = jnp.concatenate(src_list, axis=0)
    mask = jnp.concatenate(mask_list, axis=0)
    dst = jnp.broadcast_to(jnp.arange(N)[:, None], src.shape)
    edge_index = jnp.stack([src.reshape(-1), dst.reshape(-1)])
    return edge_index, mask.reshape(-1)


def _segment_mean(data, seg, num):
    s = jax.ops.segment_sum(data, seg, num_segments=num)
    cnt = jax.ops.segment_sum(jnp.ones((data.shape[0], 1), data.dtype), seg, num_segments=num)
    return s / jnp.maximum(cnt, 1.0)


def _out_proj_kernel(h_ref, w_ref, b_ref, o_ref):
    o_ref[...] = h_ref[...] @ w_ref[...] + b_ref[...]


def kernel(coords, batch, t, w_fourier, params):
    edge_index, edge_mask = _build_radius_graph(coords, batch, RADIUS, MAX_K)
    N = coords.shape[0]
    num_graphs = t.shape[0]
    x_proj = 2.0 * jnp.pi * t[:, None] * w_fourier[None, :]
    ff = jnp.concatenate([jnp.sin(x_proj), jnp.cos(x_proj)], axis=-1)
    t_feat = _silu(ff @ params['t_embed_W'] + params['t_embed_b'])
    h = coords @ params['in_W'] + params['in_b']
    src = edge_index[0]
    dst = edge_index[1]
    edge_vec = jnp.where(edge_mask[:, None], coords[src] - coords[dst], 1.0)
    norm = jnp.sqrt(jnp.sum(edge_vec ** 2, axis=-1, keepdims=True))
    edge_dir = edge_vec / jnp.maximum(norm, 1e-12)
    edge_length = norm[:, 0]
    values = jnp.linspace(0.0, RADIUS, NUM_BASIS + 2)
    step = values[1] - values[0]
    values = values[1:-1]
    diff = (edge_length[:, None] - values[None, :]) / step
    edge_scalar = jnp.exp(-diff ** 2) / 1.12
    edge_attr = jnp.concatenate([edge_scalar, edge_dir], axis=-1)
    maskf = edge_mask.astype(h.dtype)[:, None]
    for l in range(NUM_LAYERS):
        p = params['layer_%d' % l]
        t_per_node = (t_feat @ p['t_W'] + p['t_b'])[batch]
        h_res = h
        x_cat = jnp.concatenate([h, t_per_node], axis=-1)
        edge_emb = edge_attr @ p['edge_W'] + p['edge_b']
        msg = jax.nn.relu(x_cat[src] + edge_emb) * maskf
        aggr = jax.ops.segment_sum(msg, dst, num_segments=N)
        out = (1.0 + p['eps']) * x_cat + aggr
        out = _silu(out @ p['mlp_W1'] + p['mlp_b1']) @ p['mlp_W2'] + p['mlp_b2']
        mean = _segment_mean(out, batch, num_graphs)[batch]
        o = out - mean * p['gn_mean_scale']
        var = _segment_mean(o ** 2, batch, num_graphs)[batch]
        o = p['gn_weight'] * o / jnp.sqrt(var + 1e-5) + p['gn_bias']
        h = _silu(o + h_res)
    out = pl.pallas_call(
        _out_proj_kernel,
        out_shape=jax.ShapeDtypeStruct((N, 3), jnp.float32),
    )(h, params['out_W'], params['out_b'])
    return out


# speedup vs baseline: 1.5097x; 1.5097x over previous
import functools

import jax
import jax.numpy as jnp
import numpy as np
from jax import lax
from jax.experimental import pallas as pl
from jax.experimental.pallas import tpu as pltpu
from jax.experimental.pallas import tpu_sc as plsc

N_NODES = 10000
N_GRAPHS = 64
HIDDEN = 256
T_EMBED = 128
NUM_LAYERS = 5
MAX_K = 30
RADIUS = 1.5
NUM_BASIS = 32
EDGE_DIM = NUM_BASIS + 3

NP = 10240
BLK = 128
NB = NP // BLK
E_PAD = NP * MAX_K
NW = 32
CH = 128
PER_W = E_PAD // NW
N_IT = PER_W // CH

HI = lax.Precision.HIGHEST
F32 = jnp.float32


def _silu(x):
    return x * jax.nn.sigmoid(x)


def _build_radius_graph(coords, batch, r, max_k, chunk=2000):
    N = coords.shape[0]
    sq = jnp.sum(coords ** 2, axis=-1)
    src_list, mask_list = [], []
    for s in range(0, N, chunk):
        c = coords[s:s + chunk]
        b = batch[s:s + chunk]
        m = c.shape[0]
        d2 = jnp.sum(c ** 2, axis=-1)[:, None] + sq[None, :] - 2.0 * (c @ coords.T)
        same = b[:, None] == batch[None, :]
        self_m = jnp.arange(N)[None, :] == (s + jnp.arange(m))[:, None]
        valid = same & (~self_m) & (d2 <= r * r)
        d2m = jnp.where(valid, d2, jnp.inf)
        vals, idxs = jax.lax.top_k(-d2m, max_k)
        src_list.append(idxs)
        mask_list.append(vals > -jnp.inf)
    src = jnp.concatenate(src_list, axis=0)
    mask = jnp.concatenate(mask_list, axis=0)
    return src, mask


def _sc_gather(table, idx):
    mesh = plsc.VectorSubcoreMesh(core_axis_name="c", subcore_axis_name="s")

    @functools.partial(
        pl.kernel,
        mesh=mesh,
        out_type=jax.ShapeDtypeStruct((E_PAD, HIDDEN), F32),
        scratch_types=[
            pltpu.VMEM((CH,), jnp.int32),
            pltpu.VMEM((CH, HIDDEN), F32),
            pltpu.SemaphoreType.DMA,
        ],
    )
    def gk(table_hbm, idx_hbm, out_hbm, idx_v, rows_v, sem):
        wid = lax.axis_index("s") * 2 + lax.axis_index("c")
        base = wid * PER_W

        def body(i, carry):
            off = base + i * CH
            pltpu.sync_copy(idx_hbm.at[pl.ds(off, CH)], idx_v)
            pltpu.async_copy(table_hbm.at[idx_v], rows_v, sem).wait()
            pltpu.sync_copy(rows_v, out_hbm.at[pl.ds(off, CH)])
            return carry

        lax.fori_loop(0, N_IT, body, 0)

    return gk(table, idx)


def _dot(a, b):
    return jnp.dot(a, b, precision=HI, preferred_element_type=F32)


def _dotb(a, b):
    return jnp.dot(a.astype(jnp.bfloat16), b.astype(jnp.bfloat16),
                   preferred_element_type=F32)


def _temb_body(t_ref, wf_ref, W_ref, b_ref, tW_ref, tb_ref, out_ref):
    xp = (2.0 * jnp.pi) * t_ref[...] * wf_ref[...]
    ff = jnp.concatenate([jnp.sin(xp), jnp.cos(xp)], axis=-1)
    tf = _dotb(ff, W_ref[...]) + b_ref[...]
    tf = _silu(tf)
    for l in range(NUM_LAYERS):
        out_ref[l] = _dotb(tf, tW_ref[l]) + tb_ref[l]


def _h0_body(c_ref, W_ref, b_ref, o_ref):
    o_ref[...] = _dotb(c_ref[...], W_ref[...]) + b_ref[...]


def _msg_body(g_ref, ea_ref, m_ref, h_ref, oh_ref, tall_ref, ew_ref, eb_ref,
              W1h_ref, W1t_ref, b1_ref, W2_ref, b2_ref, eps_ref, u_ref):
    th = _dot(oh_ref[...], tall_ref[...])
    hh = h_ref[...]
    acc_h = jnp.zeros((BLK, HIDDEN), F32)
    acc_t = jnp.zeros((BLK, HIDDEN), F32)
    for k in range(MAX_K):
        e_k = _dotb(ea_ref[k], ew_ref[...]) + eb_ref[...]
        mk = m_ref[:, k:k + 1]
        acc_h = acc_h + jnp.maximum(g_ref[k] + e_k[:, :HIDDEN], 0.0) * mk
        acc_t = acc_t + jnp.maximum(th + e_k[:, HIDDEN:], 0.0) * mk
    ep = 1.0 + eps_ref[0, 0]
    oh = ep * hh + acc_h
    ot = ep * th + acc_t
    z = _dotb(oh, W1h_ref[...]) + _dotb(ot, W1t_ref[...]) + b1_ref[...]
    z = _silu(z)
    u_ref[...] = _dotb(z, W2_ref[...]) + b2_ref[...]


def _gsum_body(ohT_ref, u_ref, su_ref, suu_ref):
    i = pl.program_id(0)

    @pl.when(i == 0)
    def _():
        su_ref[...] = jnp.zeros_like(su_ref)
        suu_ref[...] = jnp.zeros_like(suu_ref)

    ohT = ohT_ref[...]
    u = u_ref[...]
    su_ref[...] += _dot(ohT, u)
    suu_ref[...] += _dot(ohT, u * u)


def _norm_body(u_ref, h_ref, oh_ref, A_ref, R_ref, gnb_ref, o_ref):
    A = _dot(oh_ref[...], A_ref[...])
    R = _dot(oh_ref[...], R_ref[...])
    o = R * (u_ref[...] - A) + gnb_ref[...] + h_ref[...]
    o_ref[...] = _silu(o)


def _out_body(h_ref, W_ref, b_ref, o_ref):
    o_ref[...] = _dotb(h_ref[...], W_ref[...]) + b_ref[...]


def _row_spec():
    return pl.BlockSpec((BLK, HIDDEN), lambda i: (i, 0))


def _full_spec(shape):
    nd = len(shape)
    return pl.BlockSpec(shape, lambda i: (0,) * nd)


def kernel(coords, batch, t, w_fourier, params):
    N = coords.shape[0]
    src, mask = _build_radius_graph(coords, batch, RADIUS, MAX_K)

    srcf = src.reshape(-1)
    dstf = jnp.broadcast_to(jnp.arange(N)[:, None], src.shape).reshape(-1)
    maskf = mask.reshape(-1)
    edge_vec = jnp.where(maskf[:, None], coords[srcf] - coords[dstf], 1.0)
    norm = jnp.sqrt(jnp.sum(edge_vec ** 2, axis=-1, keepdims=True))
    edge_dir = edge_vec / jnp.maximum(norm, 1e-12)
    edge_length = norm[:, 0]
    values = jnp.linspace(0.0, RADIUS, NUM_BASIS + 2)
    step = values[1] - values[0]
    values = values[1:-1]
    diff = (edge_length[:, None] - values[None, :]) / step
    edge_scalar = jnp.exp(-diff ** 2) / 1.12
    edge_attr = jnp.concatenate([edge_scalar, edge_dir], axis=-1)

    pad_n = NP - N
    ea3 = edge_attr.reshape(N, MAX_K, EDGE_DIM).transpose(1, 0, 2)
    ea3 = jnp.pad(ea3, ((0, 0), (0, pad_n), (0, 0)))
    m2 = jnp.pad(mask.astype(F32), ((0, pad_n), (0, 2)))
    src_sm = jnp.pad(src.astype(jnp.int32).T, ((0, 0), (0, pad_n))).reshape(-1)

    batch_pad = jnp.pad(batch.astype(jnp.int32), (0, pad_n),
                        constant_values=N_GRAPHS)
    gids = jnp.arange(N_GRAPHS, dtype=jnp.int32)
    onehot = (batch_pad[:, None] == gids[None, :]).astype(F32)
    onehotT = onehot.T
    cnt = jnp.maximum(jnp.sum(onehot, axis=0), 1.0)

    coords_pad = jnp.pad(coords, ((0, pad_n), (0, 0)))

    tW_all = jnp.stack([params['layer_%d' % l]['t_W'] for l in range(NUM_LAYERS)])
    tb_all = jnp.stack([params['layer_%d' % l]['t_b'] for l in range(NUM_LAYERS)])[:, None, :]
    t_all = pl.pallas_call(
        _temb_body,
        out_shape=jax.ShapeDtypeStruct((NUM_LAYERS, N_GRAPHS, HIDDEN), F32),
    )(t.reshape(N_GRAPHS, 1), w_fourier.reshape(1, T_EMBED // 2),
      params['t_embed_W'], params['t_embed_b'].reshape(1, T_EMBED),
      tW_all, tb_all)

    h = pl.pallas_call(
        _h0_body,
        out_shape=jax.ShapeDtypeStruct((NP, HIDDEN), F32),
    )(coords_pad, params['in_W'], params['in_b'].reshape(1, HIDDEN))

    for l in range(NUM_LAYERS):
        p = params['layer_%d' % l]
        gathered = _sc_gather(h, src_sm)
        g3 = gathered.reshape(MAX_K, NP, HIDDEN)

        u = pl.pallas_call(
            _msg_body,
            grid=(NB,),
            in_specs=[
                pl.BlockSpec((MAX_K, BLK, HIDDEN), lambda i: (0, i, 0)),
                pl.BlockSpec((MAX_K, BLK, EDGE_DIM), lambda i: (0, i, 0)),
                pl.BlockSpec((BLK, 32), lambda i: (i, 0)),
                _row_spec(),
                pl.BlockSpec((BLK, N_GRAPHS), lambda i: (i, 0)),
                _full_spec((N_GRAPHS, HIDDEN)),
                _full_spec((EDGE_DIM, 2 * HIDDEN)),
                _full_spec((1, 2 * HIDDEN)),
                _full_spec((HIDDEN, HIDDEN)),
                _full_spec((HIDDEN, HIDDEN)),
                _full_spec((1, HIDDEN)),
                _full_spec((HIDDEN, HIDDEN)),
                _full_spec((1, HIDDEN)),
                _full_spec((1, 1)),
            ],
            out_specs=_row_spec(),
            out_shape=jax.ShapeDtypeStruct((NP, HIDDEN), F32),
        )(g3, ea3, m2, h, onehot, t_all[l],
          p['edge_W'], p['edge_b'].reshape(1, 2 * HIDDEN),
          p['mlp_W1'][:HIDDEN], p['mlp_W1'][HIDDEN:],
          p['mlp_b1'].reshape(1, HIDDEN),
          p['mlp_W2'], p['mlp_b2'].reshape(1, HIDDEN),
          p['eps'].reshape(1, 1))

        su, suu = pl.pallas_call(
            _gsum_body,
            grid=(NB,),
            in_specs=[
                pl.BlockSpec((N_GRAPHS, BLK), lambda i: (0, i)),
                _row_spec(),
            ],
            out_specs=[_full_spec((N_GRAPHS, HIDDEN)),
                       _full_spec((N_GRAPHS, HIDDEN))],
            out_shape=[jax.ShapeDtypeStruct((N_GRAPHS, HIDDEN), F32),
                       jax.ShapeDtypeStruct((N_GRAPHS, HIDDEN), F32)],
        )(onehotT, u)

        mean = su / cnt[:, None]
        e_uu = suu / cnt[:, None]
        s = p['gn_mean_scale'][None, :]
        var = e_uu + (s * s - 2.0 * s) * mean * mean
        A = mean * s
        R = p['gn_weight'][None, :] / jnp.sqrt(var + 1e-5)

        h = pl.pallas_call(
            _norm_body,
            grid=(NB,),
            in_specs=[
                _row_spec(),
                _row_spec(),
                pl.BlockSpec((BLK, N_GRAPHS), lambda i: (i, 0)),
                _full_spec((N_GRAPHS, HIDDEN)),
                _full_spec((N_GRAPHS, HIDDEN)),
                _full_spec((1, HIDDEN)),
            ],
            out_specs=_row_spec(),
            out_shape=jax.ShapeDtypeStruct((NP, HIDDEN), F32),
        )(u, h, onehot, A, R, p['gn_bias'].reshape(1, HIDDEN))

    out = pl.pallas_call(
        _out_body,
        out_shape=jax.ShapeDtypeStruct((NP, 3), F32),
    )(h, params['out_W'], params['out_b'].reshape(1, 3))
    return out[:N]


# speedup vs baseline: 2.0108x; 1.3320x over previous
import functools

import jax
import jax.numpy as jnp
import numpy as np
from jax import lax
from jax.experimental import pallas as pl
from jax.experimental.pallas import tpu as pltpu
from jax.experimental.pallas import tpu_sc as plsc

N_NODES = 10000
N_GRAPHS = 64
HIDDEN = 256
T_EMBED = 128
NUM_LAYERS = 5
MAX_K = 30
RADIUS = 1.5
NUM_BASIS = 32
EDGE_DIM = NUM_BASIS + 3

NP = 10240
BLK = 128
NB = NP // BLK
E_PAD = NP * MAX_K
NW = 32
CH = 128
PER_W = E_PAD // NW
N_IT = PER_W // CH

HI = lax.Precision.HIGHEST
F32 = jnp.float32


def _silu(x):
    return x * jax.nn.sigmoid(x)


GROW = 128
GCOL = 256
BIGI = np.int32(2 ** 30)


def _graph_body(st_ref, coords_ref, coordsT_ref, ssq_ref, sq_ref, bd_ref, bi_ref):
    g = pl.program_id(0)
    rs = st_ref[0, g]
    re = st_ref[0, g + 1]
    rs8 = (rs // 8) * 8
    rs128 = (rs // 128) * 128
    n_tiles = (re - rs8 + GROW - 1) // GROW
    n_chunks = (re - rs128 + GCOL - 1) // GCOL

    def tile_body(ti, _):
        r0 = pl.multiple_of(rs8 + ti * GROW, 8)
        crows = coords_ref[pl.ds(r0, GROW), :]
        ssqr = ssq_ref[pl.ds(r0, GROW), :]
        rowid = r0 + lax.broadcasted_iota(jnp.int32, (GROW, 1), 0)
        bd0 = jnp.full((GROW, 32), jnp.inf, F32)
        bi0 = jnp.full((GROW, 32), BIGI, jnp.int32)

        def chunk_body(ci, carry):
            bd, bi = carry
            cs = pl.multiple_of(rs128 + ci * GCOL, 128)
            ct = coordsT_ref[:, pl.ds(cs, GCOL)]
            mm = jnp.dot(crows.astype(jnp.bfloat16), ct.astype(jnp.bfloat16),
                         preferred_element_type=F32)
            sqc = sq_ref[:, pl.ds(cs, GCOL)]
            d2 = (ssqr + sqc) - 2.0 * mm
            colid = cs + lax.broadcasted_iota(jnp.int32, (GROW, GCOL), 1)
            valid = ((colid >= rs) & (colid < re) & (colid != rowid)
                     & (d2 <= RADIUS * RADIUS))
            d2m = jnp.where(valid, d2, jnp.inf)
            wd = jnp.concatenate([d2m, bd], axis=1)
            wi = jnp.concatenate([colid, bi], axis=1)
            nd, ni = [], []
            for _k in range(MAX_K):
                mv = jnp.min(wd, axis=1, keepdims=True)
                mi = jnp.min(jnp.where(wd == mv, wi, BIGI), axis=1, keepdims=True)
                nd.append(mv)
                ni.append(mi)
                wd = jnp.where(wi == mi, jnp.inf, wd)
            nd.append(jnp.full((GROW, 2), jnp.inf, F32))
            ni.append(jnp.full((GROW, 2), BIGI, jnp.int32))
            return (jnp.concatenate(nd, axis=1), jnp.concatenate(ni, axis=1))

        bd, bi = lax.fori_loop(0, n_chunks, chunk_body, (bd0, bi0))
        rv = (rowid >= rs) & (rowid < re)
        bd_ref[pl.ds(r0, GROW), :] = jnp.where(rv, bd, bd_ref[pl.ds(r0, GROW), :])
        bi_ref[pl.ds(r0, GROW), :] = jnp.where(rv, bi, bi_ref[pl.ds(r0, GROW), :])
        return 0

    lax.fori_loop(0, n_tiles, tile_body, 0)


def _graph_build(coords, batch):
    N = coords.shape[0]
    starts = jnp.searchsorted(batch, jnp.arange(N_GRAPHS + 1)).astype(jnp.int32)
    coords_pad = jnp.pad(coords, ((0, NP - N), (0, 0)))
    sq = jnp.sum(coords ** 2, axis=-1)
    sq_pad = jnp.pad(sq, (0, NP - N))
    bd, bi = pl.pallas_call(
        _graph_body,
        grid=(N_GRAPHS,),
        in_specs=[
            pl.BlockSpec(memory_space=pltpu.SMEM),
            pl.BlockSpec((NP, 3), lambda g: (0, 0)),
            pl.BlockSpec((3, NP), lambda g: (0, 0)),
            pl.BlockSpec((NP, 1), lambda g: (0, 0)),
            pl.BlockSpec((1, NP), lambda g: (0, 0)),
        ],
        out_specs=[pl.BlockSpec((NP, 32), lambda g: (0, 0)),
                   pl.BlockSpec((NP, 32), lambda g: (0, 0))],
        out_shape=[jax.ShapeDtypeStruct((NP, 32), F32),
                   jax.ShapeDtypeStruct((NP, 32), jnp.int32)],
    )(starts.reshape(1, -1), coords_pad, coords_pad.T,
      sq_pad.reshape(-1, 1), sq_pad.reshape(1, -1))
    lane_ok = jnp.arange(32) < MAX_K
    node_ok = jnp.arange(NP) < N
    mbool = (bd < jnp.inf) & lane_ok[None, :] & node_ok[:, None]
    srcp = jnp.where(mbool, bi, 0)
    return srcp, mbool


def _sc_gather(table, idx):
    mesh = plsc.VectorSubcoreMesh(core_axis_name="c", subcore_axis_name="s")

    @functools.partial(
        pl.kernel,
        mesh=mesh,
        out_type=jax.ShapeDtypeStruct((E_PAD, HIDDEN), F32),
        scratch_types=[
            pltpu.VMEM((CH,), jnp.int32),
            pltpu.VMEM((CH, HIDDEN), F32),
            pltpu.SemaphoreType.DMA,
        ],
    )
    def gk(table_hbm, idx_hbm, out_hbm, idx_v, rows_v, sem):
        wid = lax.axis_index("s") * 2 + lax.axis_index("c")
        base = wid * PER_W

        def body(i, carry):
            off = base + i * CH
            pltpu.sync_copy(idx_hbm.at[pl.ds(off, CH)], idx_v)
            pltpu.async_copy(table_hbm.at[idx_v], rows_v, sem).wait()
            pltpu.sync_copy(rows_v, out_hbm.at[pl.ds(off, CH)])
            return carry

        lax.fori_loop(0, N_IT, body, 0)

    return gk(table, idx)


def _dot(a, b):
    return jnp.dot(a, b, precision=HI, preferred_element_type=F32)


def _dotb(a, b):
    return jnp.dot(a.astype(jnp.bfloat16), b.astype(jnp.bfloat16),
                   preferred_element_type=F32)


def _temb_body(t_ref, wf_ref, W_ref, b_ref, tW_ref, tb_ref, out_ref):
    xp = (2.0 * jnp.pi) * t_ref[...] * wf_ref[...]
    ff = jnp.concatenate([jnp.sin(xp), jnp.cos(xp)], axis=-1)
    tf = _dotb(ff, W_ref[...]) + b_ref[...]
    tf = _silu(tf)
    for l in range(NUM_LAYERS):
        out_ref[l] = _dotb(tf, tW_ref[l]) + tb_ref[l]


def _h0_body(c_ref, W_ref, b_ref, o_ref):
    o_ref[...] = _dotb(c_ref[...], W_ref[...]) + b_ref[...]


def _msg_body(g_ref, ea_ref, m_ref, h_ref, oh_ref, tall_ref, ew_ref, eb_ref,
              W1h_ref, W1t_ref, b1_ref, W2_ref, b2_ref, eps_ref, u_ref):
    th = _dot(oh_ref[...], tall_ref[...])
    hh = h_ref[...]
    acc_h = jnp.zeros((BLK, HIDDEN), F32)
    acc_t = jnp.zeros((BLK, HIDDEN), F32)
    for k in range(MAX_K):
        e_k = _dotb(ea_ref[k], ew_ref[...]) + eb_ref[...]
        mk = m_ref[:, k:k + 1]
        acc_h = acc_h + jnp.maximum(g_ref[k] + e_k[:, :HIDDEN], 0.0) * mk
        acc_t = acc_t + jnp.maximum(th + e_k[:, HIDDEN:], 0.0) * mk
    ep = 1.0 + eps_ref[0, 0]
    oh = ep * hh + acc_h
    ot = ep * th + acc_t
    z = _dotb(oh, W1h_ref[...]) + _dotb(ot, W1t_ref[...]) + b1_ref[...]
    z = _silu(z)
    u_ref[...] = _dotb(z, W2_ref[...]) + b2_ref[...]


def _gsum_body(ohT_ref, u_ref, su_ref, suu_ref):
    i = pl.program_id(0)

    @pl.when(i == 0)
    def _():
        su_ref[...] = jnp.zeros_like(su_ref)
        suu_ref[...] = jnp.zeros_like(suu_ref)

    ohT = ohT_ref[...]
    u = u_ref[...]
    su_ref[...] += _dot(ohT, u)
    suu_ref[...] += _dot(ohT, u * u)


def _norm_body(u_ref, h_ref, oh_ref, A_ref, R_ref, gnb_ref, o_ref):
    A = _dot(oh_ref[...], A_ref[...])
    R = _dot(oh_ref[...], R_ref[...])
    o = R * (u_ref[...] - A) + gnb_ref[...] + h_ref[...]
    o_ref[...] = _silu(o)


def _out_body(h_ref, W_ref, b_ref, o_ref):
    o_ref[...] = _dotb(h_ref[...], W_ref[...]) + b_ref[...]


def _row_spec():
    return pl.BlockSpec((BLK, HIDDEN), lambda i: (i, 0))


def _full_spec(shape):
    nd = len(shape)
    return pl.BlockSpec(shape, lambda i: (0,) * nd)


def kernel(coords, batch, t, w_fourier, params):
    N = coords.shape[0]
    srcp, mboolp = _graph_build(coords, batch)
    src = srcp[:N, :MAX_K]
    mask = mboolp[:N, :MAX_K]

    srcf = src.reshape(-1)
    dstf = jnp.broadcast_to(jnp.arange(N)[:, None], src.shape).reshape(-1)
    maskf = mask.reshape(-1)
    edge_vec = jnp.where(maskf[:, None], coords[srcf] - coords[dstf], 1.0)
    norm = jnp.sqrt(jnp.sum(edge_vec ** 2, axis=-1, keepdims=True))
    edge_dir = edge_vec / jnp.maximum(norm, 1e-12)
    edge_length = norm[:, 0]
    values = jnp.linspace(0.0, RADIUS, NUM_BASIS + 2)
    step = values[1] - values[0]
    values = values[1:-1]
    diff = (edge_length[:, None] - values[None, :]) / step
    edge_scalar = jnp.exp(-diff ** 2) / 1.12
    edge_attr = jnp.concatenate([edge_scalar, edge_dir], axis=-1)

    pad_n = NP - N
    ea3 = edge_attr.reshape(N, MAX_K, EDGE_DIM).transpose(1, 0, 2)
    ea3 = jnp.pad(ea3, ((0, 0), (0, pad_n), (0, 0)))
    m2 = mboolp.astype(F32)
    src_sm = srcp[:, :MAX_K].T.reshape(-1)

    batch_pad = jnp.pad(batch.astype(jnp.int32), (0, pad_n),
                        constant_values=N_GRAPHS)
    gids = jnp.arange(N_GRAPHS, dtype=jnp.int32)
    onehot = (batch_pad[:, None] == gids[None, :]).astype(F32)
    onehotT = onehot.T
    cnt = jnp.maximum(jnp.sum(onehot, axis=0), 1.0)

    coords_pad = jnp.pad(coords, ((0, pad_n), (0, 0)))

    tW_all = jnp.stack([params['layer_%d' % l]['t_W'] for l in range(NUM_LAYERS)])
    tb_all = jnp.stack([params['layer_%d' % l]['t_b'] for l in range(NUM_LAYERS)])[:, None, :]
    t_all = pl.pallas_call(
        _temb_body,
        out_shape=jax.ShapeDtypeStruct((NUM_LAYERS, N_GRAPHS, HIDDEN), F32),
    )(t.reshape(N_GRAPHS, 1), w_fourier.reshape(1, T_EMBED // 2),
      params['t_embed_W'], params['t_embed_b'].reshape(1, T_EMBED),
      tW_all, tb_all)

    h = pl.pallas_call(
        _h0_body,
        out_shape=jax.ShapeDtypeStruct((NP, HIDDEN), F32),
    )(coords_pad, params['in_W'], params['in_b'].reshape(1, HIDDEN))

    for l in range(NUM_LAYERS):
        p = params['layer_%d' % l]
        gathered = _sc_gather(h, src_sm)
        g3 = gathered.reshape(MAX_K, NP, HIDDEN)

        u = pl.pallas_call(
            _msg_body,
            grid=(NB,),
            in_specs=[
                pl.BlockSpec((MAX_K, BLK, HIDDEN), lambda i: (0, i, 0)),
                pl.BlockSpec((MAX_K, BLK, EDGE_DIM), lambda i: (0, i, 0)),
                pl.BlockSpec((BLK, 32), lambda i: (i, 0)),
                _row_spec(),
                pl.BlockSpec((BLK, N_GRAPHS), lambda i: (i, 0)),
                _full_spec((N_GRAPHS, HIDDEN)),
                _full_spec((EDGE_DIM, 2 * HIDDEN)),
                _full_spec((1, 2 * HIDDEN)),
                _full_spec((HIDDEN, HIDDEN)),
                _full_spec((HIDDEN, HIDDEN)),
                _full_spec((1, HIDDEN)),
                _full_spec((HIDDEN, HIDDEN)),
                _full_spec((1, HIDDEN)),
                _full_spec((1, 1)),
            ],
            out_specs=_row_spec(),
            out_shape=jax.ShapeDtypeStruct((NP, HIDDEN), F32),
        )(g3, ea3, m2, h, onehot, t_all[l],
          p['edge_W'], p['edge_b'].reshape(1, 2 * HIDDEN),
          p['mlp_W1'][:HIDDEN], p['mlp_W1'][HIDDEN:],
          p['mlp_b1'].reshape(1, HIDDEN),
          p['mlp_W2'], p['mlp_b2'].reshape(1, HIDDEN),
          p['eps'].reshape(1, 1))

        su, suu = pl.pallas_call(
            _gsum_body,
            grid=(NB,),
            in_specs=[
                pl.BlockSpec((N_GRAPHS, BLK), lambda i: (0, i)),
                _row_spec(),
            ],
            out_specs=[_full_spec((N_GRAPHS, HIDDEN)),
                       _full_spec((N_GRAPHS, HIDDEN))],
            out_shape=[jax.ShapeDtypeStruct((N_GRAPHS, HIDDEN), F32),
                       jax.ShapeDtypeStruct((N_GRAPHS, HIDDEN), F32)],
        )(onehotT, u)

        mean = su / cnt[:, None]
        e_uu = suu / cnt[:, None]
        s = p['gn_mean_scale'][None, :]
        var = e_uu + (s * s - 2.0 * s) * mean * mean
        A = mean * s
        R = p['gn_weight'][None, :] / jnp.sqrt(var + 1e-5)

        h = pl.pallas_call(
            _norm_body,
            grid=(NB,),
            in_specs=[
                _row_spec(),
                _row_spec(),
                pl.BlockSpec((BLK, N_GRAPHS), lambda i: (i, 0)),
                _full_spec((N_GRAPHS, HIDDEN)),
                _full_spec((N_GRAPHS, HIDDEN)),
                _full_spec((1, HIDDEN)),
            ],
            out_specs=_row_spec(),
            out_shape=jax.ShapeDtypeStruct((NP, HIDDEN), F32),
        )(u, h, onehot, A, R, p['gn_bias'].reshape(1, HIDDEN))

    out = pl.pallas_call(
        _out_body,
        out_shape=jax.ShapeDtypeStruct((NP, 3), F32),
    )(h, params['out_W'], params['out_b'].reshape(1, 3))
    return out[:N]


# speedup vs baseline: 5.4759x; 2.7232x over previous
import functools

import jax
import jax.numpy as jnp
import numpy as np
from jax import lax
from jax.experimental import pallas as pl
from jax.experimental.pallas import tpu as pltpu
from jax.experimental.pallas import tpu_sc as plsc

N_NODES = 10000
N_GRAPHS = 64
HIDDEN = 256
T_EMBED = 128
NUM_LAYERS = 5
MAX_K = 30
RADIUS = 1.5
NUM_BASIS = 32
EDGE_DIM = NUM_BASIS + 3

NP = 10240
BLK = 128
NB = NP // BLK
E_PAD = NP * MAX_K
NW = 32
CH = 128
PER_W = E_PAD // NW
N_IT = PER_W // CH

HI = lax.Precision.HIGHEST
F32 = jnp.float32


def _silu(x):
    return x * jax.nn.sigmoid(x)


GROW = 128
GCOL = 256
BIGI = np.int32(2 ** 30)


def _graph_body(st_ref, coords_ref, coordsT_ref, ssq_ref, sq_ref, bd_ref, bi_ref):
    g = pl.program_id(0)
    rs = st_ref[0, g]
    re = st_ref[0, g + 1]
    rs8 = (rs // 8) * 8
    rs128 = (rs // 128) * 128
    n_tiles = (re - rs8 + GROW - 1) // GROW
    n_chunks = (re - rs128 + GCOL - 1) // GCOL

    def tile_body(ti, _):
        r0 = pl.multiple_of(rs8 + ti * GROW, 8)
        crows = coords_ref[pl.ds(r0, GROW), :]
        ssqr = ssq_ref[pl.ds(r0, GROW), :]
        rowid = r0 + lax.broadcasted_iota(jnp.int32, (GROW, 1), 0)
        bd0 = jnp.full((GROW, 32), jnp.inf, F32)
        bi0 = jnp.full((GROW, 32), BIGI, jnp.int32)

        def chunk_body(ci, carry):
            bd, bi = carry
            cs = pl.multiple_of(rs128 + ci * GCOL, 128)
            ct = coordsT_ref[:, pl.ds(cs, GCOL)]
            mm = jnp.dot(crows.astype(jnp.bfloat16), ct.astype(jnp.bfloat16),
                         preferred_element_type=F32)
            sqc = sq_ref[:, pl.ds(cs, GCOL)]
            d2 = (ssqr + sqc) - 2.0 * mm
            colid = cs + lax.broadcasted_iota(jnp.int32, (GROW, GCOL), 1)
            valid = ((colid >= rs) & (colid < re) & (colid != rowid)
                     & (d2 <= RADIUS * RADIUS))
            d2m = jnp.where(valid, d2, jnp.inf)
            wd = jnp.concatenate([d2m, bd], axis=1)
            wi = jnp.concatenate([colid, bi], axis=1)
            nd, ni = [], []
            for _k in range(MAX_K):
                mv = jnp.min(wd, axis=1, keepdims=True)
                mi = jnp.min(jnp.where(wd == mv, wi, BIGI), axis=1, keepdims=True)
                nd.append(mv)
                ni.append(mi)
                wd = jnp.where(wi == mi, jnp.inf, wd)
            nd.append(jnp.full((GROW, 2), jnp.inf, F32))
            ni.append(jnp.full((GROW, 2), BIGI, jnp.int32))
            return (jnp.concatenate(nd, axis=1), jnp.concatenate(ni, axis=1))

        bd, bi = lax.fori_loop(0, n_chunks, chunk_body, (bd0, bi0))
        rv = (rowid >= rs) & (rowid < re)
        bd_ref[pl.ds(r0, GROW), :] = jnp.where(rv, bd, bd_ref[pl.ds(r0, GROW), :])
        bi_ref[pl.ds(r0, GROW), :] = jnp.where(rv, bi, bi_ref[pl.ds(r0, GROW), :])
        return 0

    lax.fori_loop(0, n_tiles, tile_body, 0)


def _graph_build(coords, batch):
    N = coords.shape[0]
    starts = jnp.searchsorted(batch, jnp.arange(N_GRAPHS + 1)).astype(jnp.int32)
    coords_pad = jnp.pad(coords, ((0, NP - N), (0, 0)))
    sq = jnp.sum(coords ** 2, axis=-1)
    sq_pad = jnp.pad(sq, (0, NP - N))
    bd, bi = pl.pallas_call(
        _graph_body,
        grid=(N_GRAPHS,),
        in_specs=[
            pl.BlockSpec(memory_space=pltpu.SMEM),
            pl.BlockSpec((NP, 3), lambda g: (0, 0)),
            pl.BlockSpec((3, NP), lambda g: (0, 0)),
            pl.BlockSpec((NP, 1), lambda g: (0, 0)),
            pl.BlockSpec((1, NP), lambda g: (0, 0)),
        ],
        out_specs=[pl.BlockSpec((NP, 32), lambda g: (0, 0)),
                   pl.BlockSpec((NP, 32), lambda g: (0, 0))],
        out_shape=[jax.ShapeDtypeStruct((NP, 32), F32),
                   jax.ShapeDtypeStruct((NP, 32), jnp.int32)],
    )(starts.reshape(1, -1), coords_pad, coords_pad.T,
      sq_pad.reshape(-1, 1), sq_pad.reshape(1, -1))
    lane_ok = jnp.arange(32) < MAX_K
    node_ok = jnp.arange(NP) < N
    mbool = (bd < jnp.inf) & lane_ok[None, :] & node_ok[:, None]
    own = jnp.arange(NP, dtype=jnp.int32)[:, None]
    srcp = jnp.where(mbool, bi, own)
    return srcp, mbool


def _sc_gather(table, idx):
    mesh = plsc.VectorSubcoreMesh(core_axis_name="c", subcore_axis_name="s")

    @functools.partial(
        pl.kernel,
        mesh=mesh,
        out_type=jax.ShapeDtypeStruct((E_PAD, HIDDEN), F32),
        scratch_types=[
            pltpu.VMEM((CH,), jnp.int32),
            pltpu.VMEM((CH, HIDDEN), F32),
            pltpu.SemaphoreType.DMA,
        ],
    )
    def gk(table_hbm, idx_hbm, out_hbm, idx_v, rows_v, sem):
        wid = lax.axis_index("s") * 2 + lax.axis_index("c")
        base = wid * PER_W

        def body(i, carry):
            off = base + i * CH
            pltpu.sync_copy(idx_hbm.at[pl.ds(off, CH)], idx_v)
            pltpu.async_copy(table_hbm.at[idx_v], rows_v, sem).wait()
            pltpu.sync_copy(rows_v, out_hbm.at[pl.ds(off, CH)])
            return carry

        lax.fori_loop(0, N_IT, body, 0)

    return gk(table, idx)


def _dot(a, b):
    return jnp.dot(a, b, precision=HI, preferred_element_type=F32)


def _dotb(a, b):
    return jnp.dot(a.astype(jnp.bfloat16), b.astype(jnp.bfloat16),
                   preferred_element_type=F32)


def _temb_body(t_ref, wf_ref, W_ref, b_ref, tW_ref, tb_ref, out_ref):
    xp = (2.0 * jnp.pi) * t_ref[...] * wf_ref[...]
    ff = jnp.concatenate([jnp.sin(xp), jnp.cos(xp)], axis=-1)
    tf = _dotb(ff, W_ref[...]) + b_ref[...]
    tf = _silu(tf)
    for l in range(NUM_LAYERS):
        out_ref[l] = _dotb(tf, tW_ref[l]) + tb_ref[l]


def _h0_body(c_ref, W_ref, b_ref, o_ref):
    o_ref[...] = _dotb(c_ref[...], W_ref[...]) + b_ref[...]


def _msg_body(g_ref, ea_ref, m_ref, h_ref, oh_ref, tall_ref, ew_ref, eb_ref,
              W1h_ref, W1t_ref, b1_ref, W2_ref, b2_ref, eps_ref, u_ref):
    th = _dot(oh_ref[...], tall_ref[...])
    hh = h_ref[...]
    acc_h = jnp.zeros((BLK, HIDDEN), F32)
    acc_t = jnp.zeros((BLK, HIDDEN), F32)
    for k in range(MAX_K):
        e_k = _dotb(ea_ref[k], ew_ref[...]) + eb_ref[...]
        mk = m_ref[:, k:k + 1]
        acc_h = acc_h + jnp.maximum(g_ref[k] + e_k[:, :HIDDEN], 0.0) * mk
        acc_t = acc_t + jnp.maximum(th + e_k[:, HIDDEN:], 0.0) * mk
    ep = 1.0 + eps_ref[0, 0]
    oh = ep * hh + acc_h
    ot = ep * th + acc_t
    z = _dotb(oh, W1h_ref[...]) + _dotb(ot, W1t_ref[...]) + b1_ref[...]
    z = _silu(z)
    u_ref[...] = _dotb(z, W2_ref[...]) + b2_ref[...]


def _gsum_body(ohT_ref, u_ref, su_ref, suu_ref):
    i = pl.program_id(0)

    @pl.when(i == 0)
    def _():
        su_ref[...] = jnp.zeros_like(su_ref)
        suu_ref[...] = jnp.zeros_like(suu_ref)

    ohT = ohT_ref[...]
    u = u_ref[...]
    su_ref[...] += _dot(ohT, u)
    suu_ref[...] += _dot(ohT, u * u)


def _norm_body(u_ref, h_ref, oh_ref, A_ref, R_ref, gnb_ref, o_ref):
    A = _dot(oh_ref[...], A_ref[...])
    R = _dot(oh_ref[...], R_ref[...])
    o = R * (u_ref[...] - A) + gnb_ref[...] + h_ref[...]
    o_ref[...] = _silu(o)


def _out_body(h_ref, W_ref, b_ref, o_ref):
    o_ref[...] = _dotb(h_ref[...], W_ref[...]) + b_ref[...]


def _row_spec():
    return pl.BlockSpec((BLK, HIDDEN), lambda i: (i, 0))


def _full_spec(shape):
    nd = len(shape)
    return pl.BlockSpec(shape, lambda i: (0,) * nd)


def kernel(coords, batch, t, w_fourier, params):
    N = coords.shape[0]
    srcp, mboolp = _graph_build(coords, batch)
    src = srcp[:N, :MAX_K]
    mask = mboolp[:N, :MAX_K]

    srcf = src.reshape(-1)
    dstf = jnp.broadcast_to(jnp.arange(N)[:, None], src.shape).reshape(-1)
    maskf = mask.reshape(-1)
    edge_vec = jnp.where(maskf[:, None], coords[srcf] - coords[dstf], 1.0)
    norm = jnp.sqrt(jnp.sum(edge_vec ** 2, axis=-1, keepdims=True))
    edge_dir = edge_vec / jnp.maximum(norm, 1e-12)
    edge_length = norm[:, 0]
    values = jnp.linspace(0.0, RADIUS, NUM_BASIS + 2)
    step = values[1] - values[0]
    values = values[1:-1]
    diff = (edge_length[:, None] - values[None, :]) / step
    edge_scalar = jnp.exp(-diff ** 2) / 1.12
    edge_attr = jnp.concatenate([edge_scalar, edge_dir], axis=-1)

    pad_n = NP - N
    ea3 = edge_attr.reshape(N, MAX_K, EDGE_DIM).transpose(1, 0, 2)
    ea3 = jnp.pad(ea3, ((0, 0), (0, pad_n), (0, 0)))
    m2 = mboolp.astype(F32)
    src_sm = srcp[:, :MAX_K].T.reshape(-1)

    batch_pad = jnp.pad(batch.astype(jnp.int32), (0, pad_n),
                        constant_values=N_GRAPHS)
    gids = jnp.arange(N_GRAPHS, dtype=jnp.int32)
    onehot = (batch_pad[:, None] == gids[None, :]).astype(F32)
    onehotT = onehot.T
    cnt = jnp.maximum(jnp.sum(onehot, axis=0), 1.0)

    coords_pad = jnp.pad(coords, ((0, pad_n), (0, 0)))

    tW_all = jnp.stack([params['layer_%d' % l]['t_W'] for l in range(NUM_LAYERS)])
    tb_all = jnp.stack([params['layer_%d' % l]['t_b'] for l in range(NUM_LAYERS)])[:, None, :]
    t_all = pl.pallas_call(
        _temb_body,
        out_shape=jax.ShapeDtypeStruct((NUM_LAYERS, N_GRAPHS, HIDDEN), F32),
    )(t.reshape(N_GRAPHS, 1), w_fourier.reshape(1, T_EMBED // 2),
      params['t_embed_W'], params['t_embed_b'].reshape(1, T_EMBED),
      tW_all, tb_all)

    h = pl.pallas_call(
        _h0_body,
        out_shape=jax.ShapeDtypeStruct((NP, HIDDEN), F32),
    )(coords_pad, params['in_W'], params['in_b'].reshape(1, HIDDEN))

    for l in range(NUM_LAYERS):
        p = params['layer_%d' % l]
        gathered = _sc_gather(h, src_sm)
        g3 = gathered.reshape(MAX_K, NP, HIDDEN)

        u = pl.pallas_call(
            _msg_body,
            grid=(NB,),
            in_specs=[
                pl.BlockSpec((MAX_K, BLK, HIDDEN), lambda i: (0, i, 0)),
                pl.BlockSpec((MAX_K, BLK, EDGE_DIM), lambda i: (0, i, 0)),
                pl.BlockSpec((BLK, 32), lambda i: (i, 0)),
                _row_spec(),
                pl.BlockSpec((BLK, N_GRAPHS), lambda i: (i, 0)),
                _full_spec((N_GRAPHS, HIDDEN)),
                _full_spec((EDGE_DIM, 2 * HIDDEN)),
                _full_spec((1, 2 * HIDDEN)),
                _full_spec((HIDDEN, HIDDEN)),
                _full_spec((HIDDEN, HIDDEN)),
                _full_spec((1, HIDDEN)),
                _full_spec((HIDDEN, HIDDEN)),
                _full_spec((1, HIDDEN)),
                _full_spec((1, 1)),
            ],
            out_specs=_row_spec(),
            out_shape=jax.ShapeDtypeStruct((NP, HIDDEN), F32),
        )(g3, ea3, m2, h, onehot, t_all[l],
          p['edge_W'], p['edge_b'].reshape(1, 2 * HIDDEN),
          p['mlp_W1'][:HIDDEN], p['mlp_W1'][HIDDEN:],
          p['mlp_b1'].reshape(1, HIDDEN),
          p['mlp_W2'], p['mlp_b2'].reshape(1, HIDDEN),
          p['eps'].reshape(1, 1))

        su, suu = pl.pallas_call(
            _gsum_body,
            grid=(NB,),
            in_specs=[
                pl.BlockSpec((N_GRAPHS, BLK), lambda i: (0, i)),
                _row_spec(),
            ],
            out_specs=[_full_spec((N_GRAPHS, HIDDEN)),
                       _full_spec((N_GRAPHS, HIDDEN))],
            out_shape=[jax.ShapeDtypeStruct((N_GRAPHS, HIDDEN), F32),
                       jax.ShapeDtypeStruct((N_GRAPHS, HIDDEN), F32)],
        )(onehotT, u)

        mean = su / cnt[:, None]
        e_uu = suu / cnt[:, None]
        s = p['gn_mean_scale'][None, :]
        var = e_uu + (s * s - 2.0 * s) * mean * mean
        A = mean * s
        R = p['gn_weight'][None, :] / jnp.sqrt(var + 1e-5)

        h = pl.pallas_call(
            _norm_body,
            grid=(NB,),
            in_specs=[
                _row_spec(),
                _row_spec(),
                pl.BlockSpec((BLK, N_GRAPHS), lambda i: (i, 0)),
                _full_spec((N_GRAPHS, HIDDEN)),
                _full_spec((N_GRAPHS, HIDDEN)),
                _full_spec((1, HIDDEN)),
            ],
            out_specs=_row_spec(),
            out_shape=jax.ShapeDtypeStruct((NP, HIDDEN), F32),
        )(u, h, onehot, A, R, p['gn_bias'].reshape(1, HIDDEN))

    out = pl.pallas_call(
        _out_body,
        out_shape=jax.ShapeDtypeStruct((NP, 3), F32),
    )(h, params['out_W'], params['out_b'].reshape(1, 3))
    return out[:N]


# speedup vs baseline: 8.5104x; 1.5542x over previous
import functools

import jax
import jax.numpy as jnp
import numpy as np
from jax import lax
from jax.experimental import pallas as pl
from jax.experimental.pallas import tpu as pltpu
from jax.experimental.pallas import tpu_sc as plsc

N_NODES = 10000
N_GRAPHS = 64
HIDDEN = 256
T_EMBED = 128
NUM_LAYERS = 5
MAX_K = 30
RADIUS = 1.5
NUM_BASIS = 32
EDGE_DIM = NUM_BASIS + 3

NP = 10240
BLK = 128
NB = NP // BLK
E_PAD = NP * MAX_K
NW = 32
CH = 128
PER_W = E_PAD // NW
N_IT = PER_W // CH

HI = lax.Precision.HIGHEST
F32 = jnp.float32


def _silu(x):
    return x * jax.nn.sigmoid(x)


GROW = 128
GCOL = 256
BIGI = np.int32(2 ** 30)


def _graph_body(st_ref, coords_ref, coordsT_ref, ssq_ref, sq_ref, bd_ref, bi_ref):
    g = pl.program_id(0)
    rs = st_ref[0, g]
    re = st_ref[0, g + 1]
    rs8 = (rs // 8) * 8
    rs128 = (rs // 128) * 128
    n_tiles = (re - rs8 + GROW - 1) // GROW
    n_chunks = (re - rs128 + GCOL - 1) // GCOL

    def tile_body(ti, _):
        r0 = pl.multiple_of(rs8 + ti * GROW, 8)
        crows = coords_ref[pl.ds(r0, GROW), :]
        ssqr = ssq_ref[pl.ds(r0, GROW), :]
        rowid = r0 + lax.broadcasted_iota(jnp.int32, (GROW, 1), 0)
        bd0 = jnp.full((GROW, 32), jnp.inf, F32)
        bi0 = jnp.full((GROW, 32), BIGI, jnp.int32)

        def chunk_body(ci, carry):
            bd, bi = carry
            cs = pl.multiple_of(rs128 + ci * GCOL, 128)
            ct = coordsT_ref[:, pl.ds(cs, GCOL)]
            mm = jnp.dot(crows.astype(jnp.bfloat16), ct.astype(jnp.bfloat16),
                         preferred_element_type=F32)
            sqc = sq_ref[:, pl.ds(cs, GCOL)]
            d2 = (ssqr + sqc) - 2.0 * mm
            colid = cs + lax.broadcasted_iota(jnp.int32, (GROW, GCOL), 1)
            valid = ((colid >= rs) & (colid < re) & (colid != rowid)
                     & (d2 <= RADIUS * RADIUS))
            d2m = jnp.where(valid, d2, jnp.inf)
            wd = jnp.concatenate([d2m, bd], axis=1)
            wi = jnp.concatenate([colid, bi], axis=1)
            nd, ni = [], []
            for _k in range(MAX_K):
                mv = jnp.min(wd, axis=1, keepdims=True)
                mi = jnp.min(jnp.where(wd == mv, wi, BIGI), axis=1, keepdims=True)
                nd.append(mv)
                ni.append(mi)
                wd = jnp.where(wi == mi, jnp.inf, wd)
            nd.append(jnp.full((GROW, 2), jnp.inf, F32))
            ni.append(jnp.full((GROW, 2), BIGI, jnp.int32))
            return (jnp.concatenate(nd, axis=1), jnp.concatenate(ni, axis=1))

        bd, bi = lax.fori_loop(0, n_chunks, chunk_body, (bd0, bi0))
        rv = (rowid >= rs) & (rowid < re)
        bd_ref[pl.ds(r0, GROW), :] = jnp.where(rv, bd, bd_ref[pl.ds(r0, GROW), :])
        bi_ref[pl.ds(r0, GROW), :] = jnp.where(rv, bi, bi_ref[pl.ds(r0, GROW), :])
        return 0

    lax.fori_loop(0, n_tiles, tile_body, 0)


def _graph_build(coords, batch):
    N = coords.shape[0]
    starts = jnp.searchsorted(batch, jnp.arange(N_GRAPHS + 1)).astype(jnp.int32)
    coords_pad = jnp.pad(coords, ((0, NP - N), (0, 0)))
    sq = jnp.sum(coords ** 2, axis=-1)
    sq_pad = jnp.pad(sq, (0, NP - N))
    bd, bi = pl.pallas_call(
        _graph_body,
        grid=(N_GRAPHS,),
        in_specs=[
            pl.BlockSpec(memory_space=pltpu.SMEM),
            pl.BlockSpec((NP, 3), lambda g: (0, 0)),
            pl.BlockSpec((3, NP), lambda g: (0, 0)),
            pl.BlockSpec((NP, 1), lambda g: (0, 0)),
            pl.BlockSpec((1, NP), lambda g: (0, 0)),
        ],
        out_specs=[pl.BlockSpec((NP, 32), lambda g: (0, 0)),
                   pl.BlockSpec((NP, 32), lambda g: (0, 0))],
        out_shape=[jax.ShapeDtypeStruct((NP, 32), F32),
                   jax.ShapeDtypeStruct((NP, 32), jnp.int32)],
    )(starts.reshape(1, -1), coords_pad, coords_pad.T,
      sq_pad.reshape(-1, 1), sq_pad.reshape(1, -1))
    lane_ok = jnp.arange(32) < MAX_K
    node_ok = jnp.arange(NP) < N
    mbool = (bd < jnp.inf) & lane_ok[None, :] & node_ok[:, None]
    own = jnp.arange(NP, dtype=jnp.int32)[:, None]
    srcp = jnp.where(mbool, bi, own)
    return srcp, mbool


def _sc_gather(table, idx, width=HIDDEN, ch=CH):
    n_it = PER_W // ch
    assert PER_W % ch == 0
    mesh = plsc.VectorSubcoreMesh(core_axis_name="c", subcore_axis_name="s")

    @functools.partial(
        pl.kernel,
        mesh=mesh,
        out_type=jax.ShapeDtypeStruct((E_PAD, width), F32),
        scratch_types=[
            pltpu.VMEM((ch,), jnp.int32),
            pltpu.VMEM((ch, width), F32),
            pltpu.SemaphoreType.DMA,
        ],
    )
    def gk(table_hbm, idx_hbm, out_hbm, idx_v, rows_v, sem):
        wid = lax.axis_index("s") * 2 + lax.axis_index("c")
        base = wid * PER_W

        def body(i, carry):
            off = base + i * ch
            pltpu.sync_copy(idx_hbm.at[pl.ds(off, ch)], idx_v)
            pltpu.async_copy(table_hbm.at[idx_v], rows_v, sem).wait()
            pltpu.sync_copy(rows_v, out_hbm.at[pl.ds(off, ch)])
            return carry

        lax.fori_loop(0, n_it, body, 0)

    return gk(table, idx)


def _dot(a, b):
    return jnp.dot(a, b, precision=HI, preferred_element_type=F32)


def _dotb(a, b):
    return jnp.dot(a.astype(jnp.bfloat16), b.astype(jnp.bfloat16),
                   preferred_element_type=F32)


def _temb_body(t_ref, wf_ref, W_ref, b_ref, tW_ref, tb_ref, out_ref):
    xp = (2.0 * jnp.pi) * t_ref[...] * wf_ref[...]
    ff = jnp.concatenate([jnp.sin(xp), jnp.cos(xp)], axis=-1)
    tf = _dotb(ff, W_ref[...]) + b_ref[...]
    tf = _silu(tf)
    for l in range(NUM_LAYERS):
        out_ref[l] = _dotb(tf, tW_ref[l]) + tb_ref[l]


def _h0_body(c_ref, W_ref, b_ref, o_ref):
    o_ref[...] = _dotb(c_ref[...], W_ref[...]) + b_ref[...]


def _edge_body(cg_ref, cd_ref, m_ref, val_ref, step_ref, ea_ref):
    cd = cd_ref[:, :8]
    vals = val_ref[...]
    step = step_ref[0, 0]
    for k in range(MAX_K):
        mk = m_ref[:, k:k + 1] > 0.0
        ev = jnp.where(mk, cg_ref[k][:, :8] - cd, 1.0)
        ev3 = ev[:, :3]
        nrm = jnp.sqrt(jnp.sum(ev3 * ev3, axis=1, keepdims=True))
        edir = ev3 / jnp.maximum(nrm, 1e-12)
        diff = (nrm - vals) / step
        scal = jnp.exp(-(diff * diff)) / 1.12
        ea_ref[k] = jnp.concatenate([scal, edir], axis=1)


def _msg_body(g_ref, ea_ref, m_ref, h_ref, oh_ref, tall_ref, ew_ref, eb_ref,
              W1h_ref, W1t_ref, b1_ref, W2_ref, b2_ref, eps_ref, u_ref):
    th = _dot(oh_ref[...], tall_ref[...])
    hh = h_ref[...]
    acc_h = jnp.zeros((BLK, HIDDEN), F32)
    acc_t = jnp.zeros((BLK, HIDDEN), F32)
    for k in range(MAX_K):
        e_k = _dotb(ea_ref[k], ew_ref[...]) + eb_ref[...]
        mk = m_ref[:, k:k + 1]
        acc_h = acc_h + jnp.maximum(g_ref[k] + e_k[:, :HIDDEN], 0.0) * mk
        acc_t = acc_t + jnp.maximum(th + e_k[:, HIDDEN:], 0.0) * mk
    ep = 1.0 + eps_ref[0, 0]
    oh = ep * hh + acc_h
    ot = ep * th + acc_t
    z = _dotb(oh, W1h_ref[...]) + _dotb(ot, W1t_ref[...]) + b1_ref[...]
    z = _silu(z)
    u_ref[...] = _dotb(z, W2_ref[...]) + b2_ref[...]


def _gsum_body(ohT_ref, u_ref, su_ref, suu_ref):
    i = pl.program_id(0)

    @pl.when(i == 0)
    def _():
        su_ref[...] = jnp.zeros_like(su_ref)
        suu_ref[...] = jnp.zeros_like(suu_ref)

    ohT = ohT_ref[...]
    u = u_ref[...]
    su_ref[...] += _dot(ohT, u)
    suu_ref[...] += _dot(ohT, u * u)


def _norm_body(u_ref, h_ref, oh_ref, A_ref, R_ref, gnb_ref, o_ref):
    A = _dot(oh_ref[...], A_ref[...])
    R = _dot(oh_ref[...], R_ref[...])
    o = R * (u_ref[...] - A) + gnb_ref[...] + h_ref[...]
    o_ref[...] = _silu(o)


def _out_body(h_ref, W_ref, b_ref, o_ref):
    o_ref[...] = _dotb(h_ref[...], W_ref[...]) + b_ref[...]


def _row_spec():
    return pl.BlockSpec((BLK, HIDDEN), lambda i: (i, 0))


def _full_spec(shape):
    nd = len(shape)
    return pl.BlockSpec(shape, lambda i: (0,) * nd)


def kernel(coords, batch, t, w_fourier, params):
    N = coords.shape[0]
    srcp, mboolp = _graph_build(coords, batch)

    pad_n = NP - N
    m2 = mboolp.astype(F32)
    src_sm = srcp[:, :MAX_K].T.reshape(-1)

    coords128 = jnp.pad(coords, ((0, pad_n), (0, 125)))
    cg = _sc_gather(coords128, src_sm, width=128, ch=384)
    cg3 = cg.reshape(MAX_K, NP, 128)
    values = jnp.linspace(0.0, RADIUS, NUM_BASIS + 2)
    step = (values[1] - values[0]).reshape(1, 1)
    vals = values[1:-1].reshape(1, NUM_BASIS)
    ea3 = pl.pallas_call(
        _edge_body,
        grid=(NB,),
        in_specs=[
            pl.BlockSpec((MAX_K, BLK, 128), lambda i: (0, i, 0)),
            pl.BlockSpec((BLK, 128), lambda i: (i, 0)),
            pl.BlockSpec((BLK, 32), lambda i: (i, 0)),
            _full_spec((1, NUM_BASIS)),
            _full_spec((1, 1)),
        ],
        out_specs=pl.BlockSpec((MAX_K, BLK, EDGE_DIM), lambda i: (0, i, 0)),
        out_shape=jax.ShapeDtypeStruct((MAX_K, NP, EDGE_DIM), F32),
    )(cg3, coords128, m2, vals, step)

    batch_pad = jnp.pad(batch.astype(jnp.int32), (0, pad_n),
                        constant_values=N_GRAPHS)
    gids = jnp.arange(N_GRAPHS, dtype=jnp.int32)
    onehot = (batch_pad[:, None] == gids[None, :]).astype(F32)
    onehotT = onehot.T
    cnt = jnp.maximum(jnp.sum(onehot, axis=0), 1.0)

    coords_pad = jnp.pad(coords, ((0, pad_n), (0, 0)))

    tW_all = jnp.stack([params['layer_%d' % l]['t_W'] for l in range(NUM_LAYERS)])
    tb_all = jnp.stack([params['layer_%d' % l]['t_b'] for l in range(NUM_LAYERS)])[:, None, :]
    t_all = pl.pallas_call(
        _temb_body,
        out_shape=jax.ShapeDtypeStruct((NUM_LAYERS, N_GRAPHS, HIDDEN), F32),
    )(t.reshape(N_GRAPHS, 1), w_fourier.reshape(1, T_EMBED // 2),
      params['t_embed_W'], params['t_embed_b'].reshape(1, T_EMBED),
      tW_all, tb_all)

    h = pl.pallas_call(
        _h0_body,
        out_shape=jax.ShapeDtypeStruct((NP, HIDDEN), F32),
    )(coords_pad, params['in_W'], params['in_b'].reshape(1, HIDDEN))

    for l in range(NUM_LAYERS):
        p = params['layer_%d' % l]
        gathered = _sc_gather(h, src_sm)
        g3 = gathered.reshape(MAX_K, NP, HIDDEN)

        u = pl.pallas_call(
            _msg_body,
            grid=(NB,),
            in_specs=[
                pl.BlockSpec((MAX_K, BLK, HIDDEN), lambda i: (0, i, 0)),
                pl.BlockSpec((MAX_K, BLK, EDGE_DIM), lambda i: (0, i, 0)),
                pl.BlockSpec((BLK, 32), lambda i: (i, 0)),
                _row_spec(),
                pl.BlockSpec((BLK, N_GRAPHS), lambda i: (i, 0)),
                _full_spec((N_GRAPHS, HIDDEN)),
                _full_spec((EDGE_DIM, 2 * HIDDEN)),
                _full_spec((1, 2 * HIDDEN)),
                _full_spec((HIDDEN, HIDDEN)),
                _full_spec((HIDDEN, HIDDEN)),
                _full_spec((1, HIDDEN)),
                _full_spec((HIDDEN, HIDDEN)),
                _full_spec((1, HIDDEN)),
                _full_spec((1, 1)),
            ],
            out_specs=_row_spec(),
            out_shape=jax.ShapeDtypeStruct((NP, HIDDEN), F32),
        )(g3, ea3, m2, h, onehot, t_all[l],
          p['edge_W'], p['edge_b'].reshape(1, 2 * HIDDEN),
          p['mlp_W1'][:HIDDEN], p['mlp_W1'][HIDDEN:],
          p['mlp_b1'].reshape(1, HIDDEN),
          p['mlp_W2'], p['mlp_b2'].reshape(1, HIDDEN),
          p['eps'].reshape(1, 1))

        su, suu = pl.pallas_call(
            _gsum_body,
            grid=(NB,),
            in_specs=[
                pl.BlockSpec((N_GRAPHS, BLK), lambda i: (0, i)),
                _row_spec(),
            ],
            out_specs=[_full_spec((N_GRAPHS, HIDDEN)),
                       _full_spec((N_GRAPHS, HIDDEN))],
            out_shape=[jax.ShapeDtypeStruct((N_GRAPHS, HIDDEN), F32),
                       jax.ShapeDtypeStruct((N_GRAPHS, HIDDEN), F32)],
        )(onehotT, u)

        mean = su / cnt[:, None]
        e_uu = suu / cnt[:, None]
        s = p['gn_mean_scale'][None, :]
        var = e_uu + (s * s - 2.0 * s) * mean * mean
        A = mean * s
        R = p['gn_weight'][None, :] / jnp.sqrt(var + 1e-5)

        h = pl.pallas_call(
            _norm_body,
            grid=(NB,),
            in_specs=[
                _row_spec(),
                _row_spec(),
                pl.BlockSpec((BLK, N_GRAPHS), lambda i: (i, 0)),
                _full_spec((N_GRAPHS, HIDDEN)),
                _full_spec((N_GRAPHS, HIDDEN)),
                _full_spec((1, HIDDEN)),
            ],
            out_specs=_row_spec(),
            out_shape=jax.ShapeDtypeStruct((NP, HIDDEN), F32),
        )(u, h, onehot, A, R, p['gn_bias'].reshape(1, HIDDEN))

    out = pl.pallas_call(
        _out_body,
        out_shape=jax.ShapeDtypeStruct((NP, 3), F32),
    )(h, params['out_W'], params['out_b'].reshape(1, 3))
    return out[:N]


# speedup vs baseline: 9.1332x; 1.0732x over previous
import functools

import jax
import jax.numpy as jnp
import numpy as np
from jax import lax
from jax.experimental import pallas as pl
from jax.experimental.pallas import tpu as pltpu
from jax.experimental.pallas import tpu_sc as plsc

N_NODES = 10000
N_GRAPHS = 64
HIDDEN = 256
T_EMBED = 128
NUM_LAYERS = 5
MAX_K = 30
RADIUS = 1.5
NUM_BASIS = 32
EDGE_DIM = NUM_BASIS + 3

NP = 10240
BLK = 128
NB = NP // BLK
E_PAD = NP * MAX_K
NW = 32
CH = 192
PER_W = E_PAD // NW

HI = lax.Precision.HIGHEST
F32 = jnp.float32


def _silu(x):
    return x * jax.nn.sigmoid(x)


GROW = 128
GCOL = 256
BIGI = np.int32(2 ** 30)


def _graph_body(st_ref, coords_ref, coordsT_ref, ssq_ref, sq_ref, bd_ref, bi_ref):
    g = pl.program_id(0)
    rs = st_ref[0, g]
    re = st_ref[0, g + 1]
    rs8 = (rs // 8) * 8
    rs128 = (rs // 128) * 128
    n_tiles = (re - rs8 + GROW - 1) // GROW
    n_chunks = (re - rs128 + GCOL - 1) // GCOL

    def tile_body(ti, _):
        r0 = pl.multiple_of(rs8 + ti * GROW, 8)
        crows = coords_ref[pl.ds(r0, GROW), :]
        ssqr = ssq_ref[pl.ds(r0, GROW), :]
        rowid = r0 + lax.broadcasted_iota(jnp.int32, (GROW, 1), 0)
        bd0 = jnp.full((GROW, 32), jnp.inf, F32)
        bi0 = jnp.full((GROW, 32), BIGI, jnp.int32)

        def chunk_body(ci, carry):
            bd, bi = carry
            cs = pl.multiple_of(rs128 + ci * GCOL, 128)
            ct = coordsT_ref[:, pl.ds(cs, GCOL)]
            mm = jnp.dot(crows.astype(jnp.bfloat16), ct.astype(jnp.bfloat16),
                         preferred_element_type=F32)
            sqc = sq_ref[:, pl.ds(cs, GCOL)]
            d2 = (ssqr + sqc) - 2.0 * mm
            colid = cs + lax.broadcasted_iota(jnp.int32, (GROW, GCOL), 1)
            valid = ((colid >= rs) & (colid < re) & (colid != rowid)
                     & (d2 <= RADIUS * RADIUS))
            d2m = jnp.where(valid, d2, jnp.inf)
            wd = jnp.concatenate([d2m, bd], axis=1)
            wi = jnp.concatenate([colid, bi], axis=1)
            nd, ni = [], []
            for _k in range(MAX_K):
                mv = jnp.min(wd, axis=1, keepdims=True)
                mi = jnp.min(jnp.where(wd == mv, wi, BIGI), axis=1, keepdims=True)
                nd.append(mv)
                ni.append(mi)
                wd = jnp.where(wi == mi, jnp.inf, wd)
            nd.append(jnp.full((GROW, 2), jnp.inf, F32))
            ni.append(jnp.full((GROW, 2), BIGI, jnp.int32))
            return (jnp.concatenate(nd, axis=1), jnp.concatenate(ni, axis=1))

        bd, bi = lax.fori_loop(0, n_chunks, chunk_body, (bd0, bi0))
        rv = (rowid >= rs) & (rowid < re)
        bd_ref[pl.ds(r0, GROW), :] = jnp.where(rv, bd, bd_ref[pl.ds(r0, GROW), :])
        bi_ref[pl.ds(r0, GROW), :] = jnp.where(rv, bi, bi_ref[pl.ds(r0, GROW), :])
        return 0

    lax.fori_loop(0, n_tiles, tile_body, 0)


def _graph_build(coords, batch):
    N = coords.shape[0]
    starts = jnp.searchsorted(batch, jnp.arange(N_GRAPHS + 1)).astype(jnp.int32)
    coords_pad = jnp.pad(coords, ((0, NP - N), (0, 0)))
    sq = jnp.sum(coords ** 2, axis=-1)
    sq_pad = jnp.pad(sq, (0, NP - N))
    bd, bi = pl.pallas_call(
        _graph_body,
        grid=(N_GRAPHS,),
        in_specs=[
            pl.BlockSpec(memory_space=pltpu.SMEM),
            pl.BlockSpec((NP, 3), lambda g: (0, 0)),
            pl.BlockSpec((3, NP), lambda g: (0, 0)),
            pl.BlockSpec((NP, 1), lambda g: (0, 0)),
            pl.BlockSpec((1, NP), lambda g: (0, 0)),
        ],
        out_specs=[pl.BlockSpec((NP, 32), lambda g: (0, 0)),
                   pl.BlockSpec((NP, 32), lambda g: (0, 0))],
        out_shape=[jax.ShapeDtypeStruct((NP, 32), F32),
                   jax.ShapeDtypeStruct((NP, 32), jnp.int32)],
    )(starts.reshape(1, -1), coords_pad, coords_pad.T,
      sq_pad.reshape(-1, 1), sq_pad.reshape(1, -1))
    lane_ok = jnp.arange(32) < MAX_K
    node_ok = jnp.arange(NP) < N
    mbool = (bd < jnp.inf) & lane_ok[None, :] & node_ok[:, None]
    own = jnp.arange(NP, dtype=jnp.int32)[:, None]
    srcp = jnp.where(mbool, bi, own)
    return srcp, mbool


def _sc_gather(table, idx, width=HIDDEN, ch=CH):
    n_it = PER_W // ch
    assert PER_W % ch == 0 and n_it % 2 == 0
    mesh = plsc.VectorSubcoreMesh(core_axis_name="c", subcore_axis_name="s")

    @functools.partial(
        pl.kernel,
        mesh=mesh,
        out_type=jax.ShapeDtypeStruct((E_PAD, width), F32),
        scratch_types=[
            pltpu.VMEM((ch,), jnp.int32),
            pltpu.VMEM((ch,), jnp.int32),
            pltpu.VMEM((ch, width), F32),
            pltpu.VMEM((ch, width), F32),
            pltpu.SemaphoreType.DMA,
            pltpu.SemaphoreType.DMA,
        ],
    )
    def gk(table_hbm, idx_hbm, out_hbm, idx_v0, idx_v1, rows_v0, rows_v1,
           gsem, wsem):
        wid = lax.axis_index("s") * 2 + lax.axis_index("c")
        base = wid * PER_W
        bufs = ((idx_v0, rows_v0), (idx_v1, rows_v1))

        def body(g, carry):
            for b in range(2):
                idx_v, rows_v = bufs[b]
                off = base + (2 * g + b) * ch
                pltpu.sync_copy(idx_hbm.at[pl.ds(off, ch)], idx_v)

                @pl.when(g > 0)
                def _():
                    pltpu.make_async_copy(
                        rows_v, out_hbm.at[pl.ds(off, ch)], wsem).wait()

                pltpu.async_copy(table_hbm.at[idx_v], rows_v, gsem).wait()
                pltpu.async_copy(rows_v, out_hbm.at[pl.ds(off, ch)], wsem)
            return carry

        lax.fori_loop(0, n_it // 2, body, 0)
        for b in range(2):
            pltpu.make_async_copy(
                bufs[b][1], out_hbm.at[pl.ds(base, ch)], wsem).wait()

    return gk(table, idx)


def _dot(a, b):
    return jnp.dot(a, b, precision=HI, preferred_element_type=F32)


def _dotb(a, b):
    return jnp.dot(a.astype(jnp.bfloat16), b.astype(jnp.bfloat16),
                   preferred_element_type=F32)


def _temb_body(t_ref, wf_ref, W_ref, b_ref, tW_ref, tb_ref, out_ref):
    xp = (2.0 * jnp.pi) * t_ref[...] * wf_ref[...]
    ff = jnp.concatenate([jnp.sin(xp), jnp.cos(xp)], axis=-1)
    tf = _dotb(ff, W_ref[...]) + b_ref[...]
    tf = _silu(tf)
    for l in range(NUM_LAYERS):
        out_ref[l] = _dotb(tf, tW_ref[l]) + tb_ref[l]


def _h0_body(c_ref, W_ref, b_ref, o_ref):
    o_ref[...] = _dotb(c_ref[...], W_ref[...]) + b_ref[...]


def _edge_body(cg_ref, cd_ref, m_ref, val_ref, step_ref, ea_ref):
    cd = cd_ref[:, :8]
    vals = val_ref[...]
    step = step_ref[0, 0]
    for k in range(MAX_K):
        mk = m_ref[:, k:k + 1] > 0.0
        ev = jnp.where(mk, cg_ref[k][:, :8] - cd, 1.0)
        ev3 = ev[:, :3]
        nrm = jnp.sqrt(jnp.sum(ev3 * ev3, axis=1, keepdims=True))
        edir = ev3 / jnp.maximum(nrm, 1e-12)
        diff = (nrm - vals) / step
        scal = jnp.exp(-(diff * diff)) / 1.12
        ea_ref[k] = jnp.concatenate([scal, edir], axis=1)


def _msg_body(g_ref, ea_ref, m_ref, h_ref, oh_ref, tall_ref, ew_ref, eb_ref,
              W1h_ref, W1t_ref, b1_ref, W2_ref, b2_ref, eps_ref, u_ref):
    th = _dot(oh_ref[...], tall_ref[...])
    hh = h_ref[...]
    acc_h = jnp.zeros((BLK, HIDDEN), F32)
    acc_t = jnp.zeros((BLK, HIDDEN), F32)
    for k in range(MAX_K):
        e_k = _dotb(ea_ref[k], ew_ref[...]) + eb_ref[...]
        mk = m_ref[:, k:k + 1]
        acc_h = acc_h + jnp.maximum(g_ref[k] + e_k[:, :HIDDEN], 0.0) * mk
        acc_t = acc_t + jnp.maximum(th + e_k[:, HIDDEN:], 0.0) * mk
    ep = 1.0 + eps_ref[0, 0]
    oh = ep * hh + acc_h
    ot = ep * th + acc_t
    z = _dotb(oh, W1h_ref[...]) + _dotb(ot, W1t_ref[...]) + b1_ref[...]
    z = _silu(z)
    u_ref[...] = _dotb(z, W2_ref[...]) + b2_ref[...]


def _gsum_body(ohT_ref, u_ref, su_ref, suu_ref):
    i = pl.program_id(0)

    @pl.when(i == 0)
    def _():
        su_ref[...] = jnp.zeros_like(su_ref)
        suu_ref[...] = jnp.zeros_like(suu_ref)

    ohT = ohT_ref[...]
    u = u_ref[...]
    su_ref[...] += _dot(ohT, u)
    suu_ref[...] += _dot(ohT, u * u)


def _norm_body(u_ref, h_ref, oh_ref, A_ref, R_ref, gnb_ref, o_ref):
    A = _dot(oh_ref[...], A_ref[...])
    R = _dot(oh_ref[...], R_ref[...])
    o = R * (u_ref[...] - A) + gnb_ref[...] + h_ref[...]
    o_ref[...] = _silu(o)


def _out_body(h_ref, W_ref, b_ref, o_ref):
    o_ref[...] = _dotb(h_ref[...], W_ref[...]) + b_ref[...]


def _row_spec():
    return pl.BlockSpec((BLK, HIDDEN), lambda i: (i, 0))


def _full_spec(shape):
    nd = len(shape)
    return pl.BlockSpec(shape, lambda i: (0,) * nd)


def kernel(coords, batch, t, w_fourier, params):
    N = coords.shape[0]
    srcp, mboolp = _graph_build(coords, batch)

    pad_n = NP - N
    m2 = mboolp.astype(F32)
    src_sm = srcp[:, :MAX_K].T.reshape(-1)

    coords128 = jnp.pad(coords, ((0, pad_n), (0, 125)))
    cg = _sc_gather(coords128, src_sm, width=128, ch=192)
    cg3 = cg.reshape(MAX_K, NP, 128)
    values = jnp.linspace(0.0, RADIUS, NUM_BASIS + 2)
    step = (values[1] - values[0]).reshape(1, 1)
    vals = values[1:-1].reshape(1, NUM_BASIS)
    ea3 = pl.pallas_call(
        _edge_body,
        grid=(NB,),
        in_specs=[
            pl.BlockSpec((MAX_K, BLK, 128), lambda i: (0, i, 0)),
            pl.BlockSpec((BLK, 128), lambda i: (i, 0)),
            pl.BlockSpec((BLK, 32), lambda i: (i, 0)),
            _full_spec((1, NUM_BASIS)),
            _full_spec((1, 1)),
        ],
        out_specs=pl.BlockSpec((MAX_K, BLK, EDGE_DIM), lambda i: (0, i, 0)),
        out_shape=jax.ShapeDtypeStruct((MAX_K, NP, EDGE_DIM), F32),
    )(cg3, coords128, m2, vals, step)

    batch_pad = jnp.pad(batch.astype(jnp.int32), (0, pad_n),
                        constant_values=N_GRAPHS)
    gids = jnp.arange(N_GRAPHS, dtype=jnp.int32)
    onehot = (batch_pad[:, None] == gids[None, :]).astype(F32)
    onehotT = onehot.T
    cnt = jnp.maximum(jnp.sum(onehot, axis=0), 1.0)

    coords_pad = jnp.pad(coords, ((0, pad_n), (0, 0)))

    tW_all = jnp.stack([params['layer_%d' % l]['t_W'] for l in range(NUM_LAYERS)])
    tb_all = jnp.stack([params['layer_%d' % l]['t_b'] for l in range(NUM_LAYERS)])[:, None, :]
    t_all = pl.pallas_call(
        _temb_body,
        out_shape=jax.ShapeDtypeStruct((NUM_LAYERS, N_GRAPHS, HIDDEN), F32),
    )(t.reshape(N_GRAPHS, 1), w_fourier.reshape(1, T_EMBED // 2),
      params['t_embed_W'], params['t_embed_b'].reshape(1, T_EMBED),
      tW_all, tb_all)

    h = pl.pallas_call(
        _h0_body,
        out_shape=jax.ShapeDtypeStruct((NP, HIDDEN), F32),
    )(coords_pad, params['in_W'], params['in_b'].reshape(1, HIDDEN))

    for l in range(NUM_LAYERS):
        p = params['layer_%d' % l]
        gathered = _sc_gather(h, src_sm)
        g3 = gathered.reshape(MAX_K, NP, HIDDEN)

        u = pl.pallas_call(
            _msg_body,
            grid=(NB,),
            in_specs=[
                pl.BlockSpec((MAX_K, BLK, HIDDEN), lambda i: (0, i, 0)),
                pl.BlockSpec((MAX_K, BLK, EDGE_DIM), lambda i: (0, i, 0)),
                pl.BlockSpec((BLK, 32), lambda i: (i, 0)),
                _row_spec(),
                pl.BlockSpec((BLK, N_GRAPHS), lambda i: (i, 0)),
                _full_spec((N_GRAPHS, HIDDEN)),
                _full_spec((EDGE_DIM, 2 * HIDDEN)),
                _full_spec((1, 2 * HIDDEN)),
                _full_spec((HIDDEN, HIDDEN)),
                _full_spec((HIDDEN, HIDDEN)),
                _full_spec((1, HIDDEN)),
                _full_spec((HIDDEN, HIDDEN)),
                _full_spec((1, HIDDEN)),
                _full_spec((1, 1)),
            ],
            out_specs=_row_spec(),
            out_shape=jax.ShapeDtypeStruct((NP, HIDDEN), F32),
        )(g3, ea3, m2, h, onehot, t_all[l],
          p['edge_W'], p['edge_b'].reshape(1, 2 * HIDDEN),
          p['mlp_W1'][:HIDDEN], p['mlp_W1'][HIDDEN:],
          p['mlp_b1'].reshape(1, HIDDEN),
          p['mlp_W2'], p['mlp_b2'].reshape(1, HIDDEN),
          p['eps'].reshape(1, 1))

        su, suu = pl.pallas_call(
            _gsum_body,
            grid=(NB,),
            in_specs=[
                pl.BlockSpec((N_GRAPHS, BLK), lambda i: (0, i)),
                _row_spec(),
            ],
            out_specs=[_full_spec((N_GRAPHS, HIDDEN)),
                       _full_spec((N_GRAPHS, HIDDEN))],
            out_shape=[jax.ShapeDtypeStruct((N_GRAPHS, HIDDEN), F32),
                       jax.ShapeDtypeStruct((N_GRAPHS, HIDDEN), F32)],
        )(onehotT, u)

        mean = su / cnt[:, None]
        e_uu = suu / cnt[:, None]
        s = p['gn_mean_scale'][None, :]
        var = e_uu + (s * s - 2.0 * s) * mean * mean
        A = mean * s
        R = p['gn_weight'][None, :] / jnp.sqrt(var + 1e-5)

        h = pl.pallas_call(
            _norm_body,
            grid=(NB,),
            in_specs=[
                _row_spec(),
                _row_spec(),
                pl.BlockSpec((BLK, N_GRAPHS), lambda i: (i, 0)),
                _full_spec((N_GRAPHS, HIDDEN)),
                _full_spec((N_GRAPHS, HIDDEN)),
                _full_spec((1, HIDDEN)),
            ],
            out_specs=_row_spec(),
            out_shape=jax.ShapeDtypeStruct((NP, HIDDEN), F32),
        )(u, h, onehot, A, R, p['gn_bias'].reshape(1, HIDDEN))

    out = pl.pallas_call(
        _out_body,
        out_shape=jax.ShapeDtypeStruct((NP, 3), F32),
    )(h, params['out_W'], params['out_b'].reshape(1, 3))
    return out[:N]


# speedup vs baseline: 9.5733x; 1.0482x over previous
import functools

import jax
import jax.numpy as jnp
import numpy as np
from jax import lax
from jax.experimental import pallas as pl
from jax.experimental.pallas import tpu as pltpu
from jax.experimental.pallas import tpu_sc as plsc

N_NODES = 10000
N_GRAPHS = 64
HIDDEN = 256
T_EMBED = 128
NUM_LAYERS = 5
MAX_K = 30
RADIUS = 1.5
NUM_BASIS = 32
EDGE_DIM = NUM_BASIS + 3

NP = 10240
BLK = 128
NB = NP // BLK
E_PAD = NP * MAX_K
NW = 32
CH = 192
PER_W = E_PAD // NW

HI = lax.Precision.HIGHEST
F32 = jnp.float32


def _silu(x):
    return x * jax.nn.sigmoid(x)


GROW = 128
GCOL = 256
BIGI = np.int32(2 ** 30)


def _graph_body(st_ref, coords_ref, coordsT_ref, ssq_ref, sq_ref, bd_ref, bi_ref):
    g = pl.program_id(0)
    rs = st_ref[0, g]
    re = st_ref[0, g + 1]
    rs8 = (rs // 8) * 8
    rs128 = (rs // 128) * 128
    n_tiles = (re - rs8 + GROW - 1) // GROW
    n_chunks = (re - rs128 + GCOL - 1) // GCOL

    def tile_body(ti, _):
        r0 = pl.multiple_of(rs8 + ti * GROW, 8)
        crows = coords_ref[pl.ds(r0, GROW), :]
        ssqr = ssq_ref[pl.ds(r0, GROW), :]
        rowid = r0 + lax.broadcasted_iota(jnp.int32, (GROW, 1), 0)
        bd0 = jnp.full((GROW, 32), jnp.inf, F32)
        bi0 = jnp.full((GROW, 32), BIGI, jnp.int32)

        def chunk_body(ci, carry):
            bd, bi = carry
            cs = pl.multiple_of(rs128 + ci * GCOL, 128)
            ct = coordsT_ref[:, pl.ds(cs, GCOL)]
            mm = jnp.dot(crows.astype(jnp.bfloat16), ct.astype(jnp.bfloat16),
                         preferred_element_type=F32)
            sqc = sq_ref[:, pl.ds(cs, GCOL)]
            d2 = (ssqr + sqc) - 2.0 * mm
            colid = cs + lax.broadcasted_iota(jnp.int32, (GROW, GCOL), 1)
            valid = ((colid >= rs) & (colid < re) & (colid != rowid)
                     & (d2 <= RADIUS * RADIUS))
            d2m = jnp.where(valid, d2, jnp.inf)
            wd = jnp.concatenate([d2m, bd], axis=1)
            wi = jnp.concatenate([colid, bi], axis=1)
            nd, ni = [], []
            for _k in range(MAX_K):
                mv = jnp.min(wd, axis=1, keepdims=True)
                mi = jnp.min(jnp.where(wd == mv, wi, BIGI), axis=1, keepdims=True)
                nd.append(mv)
                ni.append(mi)
                wd = jnp.where(wi == mi, jnp.inf, wd)
            nd.append(jnp.full((GROW, 2), jnp.inf, F32))
            ni.append(jnp.full((GROW, 2), BIGI, jnp.int32))
            return (jnp.concatenate(nd, axis=1), jnp.concatenate(ni, axis=1))

        bd, bi = lax.fori_loop(0, n_chunks, chunk_body, (bd0, bi0))
        rv = (rowid >= rs) & (rowid < re)
        bd_ref[pl.ds(r0, GROW), :] = jnp.where(rv, bd, bd_ref[pl.ds(r0, GROW), :])
        bi_ref[pl.ds(r0, GROW), :] = jnp.where(rv, bi, bi_ref[pl.ds(r0, GROW), :])
        return 0

    lax.fori_loop(0, n_tiles, tile_body, 0)


def _graph_build(coords, batch):
    N = coords.shape[0]
    starts = jnp.searchsorted(batch, jnp.arange(N_GRAPHS + 1)).astype(jnp.int32)
    coords_pad = jnp.pad(coords, ((0, NP - N), (0, 0)))
    sq = jnp.sum(coords ** 2, axis=-1)
    sq_pad = jnp.pad(sq, (0, NP - N))
    bd, bi = pl.pallas_call(
        _graph_body,
        grid=(N_GRAPHS,),
        in_specs=[
            pl.BlockSpec(memory_space=pltpu.SMEM),
            pl.BlockSpec((NP, 3), lambda g: (0, 0)),
            pl.BlockSpec((3, NP), lambda g: (0, 0)),
            pl.BlockSpec((NP, 1), lambda g: (0, 0)),
            pl.BlockSpec((1, NP), lambda g: (0, 0)),
        ],
        out_specs=[pl.BlockSpec((NP, 32), lambda g: (0, 0)),
                   pl.BlockSpec((NP, 32), lambda g: (0, 0))],
        out_shape=[jax.ShapeDtypeStruct((NP, 32), F32),
                   jax.ShapeDtypeStruct((NP, 32), jnp.int32)],
    )(starts.reshape(1, -1), coords_pad, coords_pad.T,
      sq_pad.reshape(-1, 1), sq_pad.reshape(1, -1))
    lane_ok = jnp.arange(32) < MAX_K
    node_ok = jnp.arange(NP) < N
    mbool = (bd < jnp.inf) & lane_ok[None, :] & node_ok[:, None]
    own = jnp.arange(NP, dtype=jnp.int32)[:, None]
    srcp = jnp.where(mbool, bi, own)
    return srcp, mbool


def _sc_gather(table, idx, width=HIDDEN, ch=CH):
    n_it = PER_W // ch
    assert PER_W % ch == 0 and n_it % 2 == 0
    mesh = plsc.VectorSubcoreMesh(core_axis_name="c", subcore_axis_name="s")

    @functools.partial(
        pl.kernel,
        mesh=mesh,
        out_type=jax.ShapeDtypeStruct((E_PAD, width), F32),
        scratch_types=[
            pltpu.VMEM((ch,), jnp.int32),
            pltpu.VMEM((ch,), jnp.int32),
            pltpu.VMEM((ch, width), F32),
            pltpu.VMEM((ch, width), F32),
            pltpu.SemaphoreType.DMA,
            pltpu.SemaphoreType.DMA,
        ],
    )
    def gk(table_hbm, idx_hbm, out_hbm, idx_v0, idx_v1, rows_v0, rows_v1,
           gsem, wsem):
        wid = lax.axis_index("s") * 2 + lax.axis_index("c")
        base = wid * PER_W
        bufs = ((idx_v0, rows_v0), (idx_v1, rows_v1))

        def body(g, carry):
            for b in range(2):
                idx_v, rows_v = bufs[b]
                off = base + (2 * g + b) * ch
                pltpu.sync_copy(idx_hbm.at[pl.ds(off, ch)], idx_v)

                @pl.when(g > 0)
                def _():
                    pltpu.make_async_copy(
                        rows_v, out_hbm.at[pl.ds(off, ch)], wsem).wait()

                pltpu.async_copy(table_hbm.at[idx_v], rows_v, gsem).wait()
                pltpu.async_copy(rows_v, out_hbm.at[pl.ds(off, ch)], wsem)
            return carry

        lax.fori_loop(0, n_it // 2, body, 0)
        for b in range(2):
            pltpu.make_async_copy(
                bufs[b][1], out_hbm.at[pl.ds(base, ch)], wsem).wait()

    return gk(table, idx)


def _dot(a, b):
    return jnp.dot(a, b, precision=HI, preferred_element_type=F32)


def _dotb(a, b):
    return jnp.dot(a.astype(jnp.bfloat16), b.astype(jnp.bfloat16),
                   preferred_element_type=F32)


def _temb_body(t_ref, wf_ref, W_ref, b_ref, tW_ref, tb_ref, out_ref):
    xp = (2.0 * jnp.pi) * t_ref[...] * wf_ref[...]
    ff = jnp.concatenate([jnp.sin(xp), jnp.cos(xp)], axis=-1)
    tf = _dotb(ff, W_ref[...]) + b_ref[...]
    tf = _silu(tf)
    for l in range(NUM_LAYERS):
        out_ref[l] = _dotb(tf, tW_ref[l]) + tb_ref[l]


def _h0_body(c_ref, W_ref, b_ref, o_ref):
    o_ref[...] = _dotb(c_ref[...], W_ref[...]) + b_ref[...]


def _edge_body(cg_ref, cd_ref, m_ref, val_ref, step_ref, ea_ref):
    cd = cd_ref[:, :8]
    vals = val_ref[...]
    step = step_ref[0, 0]
    for k in range(MAX_K):
        mk = m_ref[:, k:k + 1] > 0.0
        ev = jnp.where(mk, cg_ref[k][:, :8] - cd, 1.0)
        ev3 = ev[:, :3]
        nrm = jnp.sqrt(jnp.sum(ev3 * ev3, axis=1, keepdims=True))
        edir = ev3 / jnp.maximum(nrm, 1e-12)
        diff = (nrm - vals) / step
        scal = jnp.exp(-(diff * diff)) / 1.12
        ea_ref[k] = jnp.concatenate([scal, edir], axis=1)


def _msg_body(g_ref, ea_ref, m_ref, h_ref, oh_ref, ohT_ref, tall_ref,
              ew_ref, eb_ref, W1h_ref, W1t_ref, b1_ref, W2_ref, b2_ref,
              eps_ref, u_ref, su_ref, suu_ref):
    th = _dot(oh_ref[...], tall_ref[...])
    hh = h_ref[...]
    ea_all = ea_ref[...].reshape(MAX_K * BLK, EDGE_DIM)
    e3 = (_dotb(ea_all, ew_ref[...]) + eb_ref[...]).reshape(
        MAX_K, BLK, 2 * HIDDEN)
    acc_h = jnp.zeros((BLK, HIDDEN), F32)
    acc_t = jnp.zeros((BLK, HIDDEN), F32)
    for k in range(MAX_K):
        e_k = e3[k]
        mk = m_ref[:, k:k + 1]
        acc_h = acc_h + jnp.maximum(g_ref[k] + e_k[:, :HIDDEN], 0.0) * mk
        acc_t = acc_t + jnp.maximum(th + e_k[:, HIDDEN:], 0.0) * mk
    ep = 1.0 + eps_ref[0, 0]
    oh = ep * hh + acc_h
    ot = ep * th + acc_t
    z = _dotb(oh, W1h_ref[...]) + _dotb(ot, W1t_ref[...]) + b1_ref[...]
    z = _silu(z)
    u = _dotb(z, W2_ref[...]) + b2_ref[...]
    u_ref[...] = u

    i = pl.program_id(0)

    @pl.when(i == 0)
    def _():
        su_ref[...] = jnp.zeros_like(su_ref)
        suu_ref[...] = jnp.zeros_like(suu_ref)

    ohT = ohT_ref[...]
    su_ref[...] += _dot(ohT, u)
    suu_ref[...] += _dot(ohT, u * u)


def _norm_body(u_ref, h_ref, oh_ref, A_ref, R_ref, gnb_ref, o_ref):
    A = _dot(oh_ref[...], A_ref[...])
    R = _dot(oh_ref[...], R_ref[...])
    o = R * (u_ref[...] - A) + gnb_ref[...] + h_ref[...]
    o_ref[...] = _silu(o)


def _out_body(h_ref, W_ref, b_ref, o_ref):
    o_ref[...] = _dotb(h_ref[...], W_ref[...]) + b_ref[...]


def _row_spec():
    return pl.BlockSpec((BLK, HIDDEN), lambda i: (i, 0))


def _full_spec(shape):
    nd = len(shape)
    return pl.BlockSpec(shape, lambda i: (0,) * nd)


def kernel(coords, batch, t, w_fourier, params):
    N = coords.shape[0]
    srcp, mboolp = _graph_build(coords, batch)

    pad_n = NP - N
    m2 = mboolp.astype(F32)
    src_sm = srcp[:, :MAX_K].T.reshape(-1)

    coords128 = jnp.pad(coords, ((0, pad_n), (0, 125)))
    cg = _sc_gather(coords128, src_sm, width=128, ch=192)
    cg3 = cg.reshape(MAX_K, NP, 128)
    values = jnp.linspace(0.0, RADIUS, NUM_BASIS + 2)
    step = (values[1] - values[0]).reshape(1, 1)
    vals = values[1:-1].reshape(1, NUM_BASIS)
    ea3 = pl.pallas_call(
        _edge_body,
        grid=(NB,),
        in_specs=[
            pl.BlockSpec((MAX_K, BLK, 128), lambda i: (0, i, 0)),
            pl.BlockSpec((BLK, 128), lambda i: (i, 0)),
            pl.BlockSpec((BLK, 32), lambda i: (i, 0)),
            _full_spec((1, NUM_BASIS)),
            _full_spec((1, 1)),
        ],
        out_specs=pl.BlockSpec((MAX_K, BLK, EDGE_DIM), lambda i: (0, i, 0)),
        out_shape=jax.ShapeDtypeStruct((MAX_K, NP, EDGE_DIM), F32),
    )(cg3, coords128, m2, vals, step)

    batch_pad = jnp.pad(batch.astype(jnp.int32), (0, pad_n),
                        constant_values=N_GRAPHS)
    gids = jnp.arange(N_GRAPHS, dtype=jnp.int32)
    onehot = (batch_pad[:, None] == gids[None, :]).astype(F32)
    onehotT = onehot.T
    cnt = jnp.maximum(jnp.sum(onehot, axis=0), 1.0)

    coords_pad = jnp.pad(coords, ((0, pad_n), (0, 0)))

    tW_all = jnp.stack([params['layer_%d' % l]['t_W'] for l in range(NUM_LAYERS)])
    tb_all = jnp.stack([params['layer_%d' % l]['t_b'] for l in range(NUM_LAYERS)])[:, None, :]
    t_all = pl.pallas_call(
        _temb_body,
        out_shape=jax.ShapeDtypeStruct((NUM_LAYERS, N_GRAPHS, HIDDEN), F32),
    )(t.reshape(N_GRAPHS, 1), w_fourier.reshape(1, T_EMBED // 2),
      params['t_embed_W'], params['t_embed_b'].reshape(1, T_EMBED),
      tW_all, tb_all)

    h = pl.pallas_call(
        _h0_body,
        out_shape=jax.ShapeDtypeStruct((NP, HIDDEN), F32),
    )(coords_pad, params['in_W'], params['in_b'].reshape(1, HIDDEN))

    for l in range(NUM_LAYERS):
        p = params['layer_%d' % l]
        gathered = _sc_gather(h, src_sm)
        g3 = gathered.reshape(MAX_K, NP, HIDDEN)

        u, su, suu = pl.pallas_call(
            _msg_body,
            grid=(NB,),
            in_specs=[
                pl.BlockSpec((MAX_K, BLK, HIDDEN), lambda i: (0, i, 0)),
                pl.BlockSpec((MAX_K, BLK, EDGE_DIM), lambda i: (0, i, 0)),
                pl.BlockSpec((BLK, 32), lambda i: (i, 0)),
                _row_spec(),
                pl.BlockSpec((BLK, N_GRAPHS), lambda i: (i, 0)),
                pl.BlockSpec((N_GRAPHS, BLK), lambda i: (0, i)),
                _full_spec((N_GRAPHS, HIDDEN)),
                _full_spec((EDGE_DIM, 2 * HIDDEN)),
                _full_spec((1, 2 * HIDDEN)),
                _full_spec((HIDDEN, HIDDEN)),
                _full_spec((HIDDEN, HIDDEN)),
                _full_spec((1, HIDDEN)),
                _full_spec((HIDDEN, HIDDEN)),
                _full_spec((1, HIDDEN)),
                _full_spec((1, 1)),
            ],
            out_specs=[_row_spec(),
                       _full_spec((N_GRAPHS, HIDDEN)),
                       _full_spec((N_GRAPHS, HIDDEN))],
            out_shape=[jax.ShapeDtypeStruct((NP, HIDDEN), F32),
                       jax.ShapeDtypeStruct((N_GRAPHS, HIDDEN), F32),
                       jax.ShapeDtypeStruct((N_GRAPHS, HIDDEN), F32)],
        )(g3, ea3, m2, h, onehot, onehotT, t_all[l],
          p['edge_W'], p['edge_b'].reshape(1, 2 * HIDDEN),
          p['mlp_W1'][:HIDDEN], p['mlp_W1'][HIDDEN:],
          p['mlp_b1'].reshape(1, HIDDEN),
          p['mlp_W2'], p['mlp_b2'].reshape(1, HIDDEN),
          p['eps'].reshape(1, 1))

        mean = su / cnt[:, None]
        e_uu = suu / cnt[:, None]
        s = p['gn_mean_scale'][None, :]
        var = e_uu + (s * s - 2.0 * s) * mean * mean
        A = mean * s
        R = p['gn_weight'][None, :] / jnp.sqrt(var + 1e-5)

        h = pl.pallas_call(
            _norm_body,
            grid=(NB,),
            in_specs=[
                _row_spec(),
                _row_spec(),
                pl.BlockSpec((BLK, N_GRAPHS), lambda i: (i, 0)),
                _full_spec((N_GRAPHS, HIDDEN)),
                _full_spec((N_GRAPHS, HIDDEN)),
                _full_spec((1, HIDDEN)),
            ],
            out_specs=_row_spec(),
            out_shape=jax.ShapeDtypeStruct((NP, HIDDEN), F32),
        )(u, h, onehot, A, R, p['gn_bias'].reshape(1, HIDDEN))

    out = pl.pallas_call(
        _out_body,
        out_shape=jax.ShapeDtypeStruct((NP, 3), F32),
    )(h, params['out_W'], params['out_b'].reshape(1, 3))
    return out[:N]
